# baseline (speedup 1.0000x reference)
import functools

import jax
import jax.numpy as jnp
import numpy as np
from jax import lax
from jax.experimental import pallas as pl
from jax.experimental.pallas import tpu as pltpu
from jax.experimental.pallas import tpu_sc as plsc

N = 10000
E = 320000
D = 128
GW = 144
NC = 2
NS = 16
NW = NC * NS
EPW = E // NW
CH = 40
SUB = 10
NCHUNK = EPW // CH
NSUPER = NCHUNK // SUB
RPT = N // NS
TROWS = 2 * N + E
NB = 2000

_PS = np.empty((D,), np.int64)
for _g in range(D // 32):
    for _k in range(16):
        _PS[_g * 32 + 2 * _k] = _g * 32 + _k
        _PS[_g * 32 + 2 * _k + 1] = _g * 32 + 16 + _k


def _pre_body(x_ref, ea_ref, w_ref, b_ref, o_ref):
    i = pl.program_id(0)

    @pl.when(i < N // NB)
    def _():
        o_ref[...] = jnp.dot(x_ref[...], w_ref[:, pl.ds(0, D)],
                             preferred_element_type=jnp.float32).astype(jnp.bfloat16)

    @pl.when(jnp.logical_and(i >= N // NB, i < 2 * (N // NB)))
    def _():
        o_ref[...] = jnp.dot(x_ref[...], w_ref[:, pl.ds(D, D)],
                             preferred_element_type=jnp.float32).astype(jnp.bfloat16)

    @pl.when(i >= 2 * (N // NB))
    def _():
        o_ref[...] = (jnp.dot(ea_ref[...], w_ref[:, pl.ds(2 * D, D)],
                              preferred_element_type=jnp.float32)
                      + b_ref[...]).astype(jnp.bfloat16)


def _node_body(g0_ref, g1_ref, x_ref, we2e_ref, wn1a_ref, wn1b_ref, bn1_ref,
               wn2_ref, bn2_ref, o_ref):
    g = g0_ref[...] + g1_ref[...]
    agg = jnp.dot(g, we2e_ref[...], preferred_element_type=jnp.float32)
    h2 = jnp.maximum(
        jnp.dot(x_ref[...], wn1a_ref[...], preferred_element_type=jnp.float32)
        + jnp.dot(agg, wn1b_ref[...], preferred_element_type=jnp.float32)
        + bn1_ref[...], 0.0)
    o_ref[...] = jnp.dot(h2, wn2_ref[...],
                         preferred_element_type=jnp.float32) + bn2_ref[...]


def _sc_body(cidx_hbm, col_hbm, tab_hbm, out_hbm,
             cidx, colb, g0, g1, h0, h1, gsh,
             sem_g0, sem_g1, sem_s0, sem_s1):
    cid = lax.axis_index("c")
    sid = lax.axis_index("s")
    wid = sid * NC + cid

    gbufs = (g0, g1)
    hbufs = (h0, h1)
    sgs = (sem_g0, sem_g1)
    sss = (sem_s0, sem_s1)

    zero16 = jnp.zeros((16,), jnp.float32)
    iota16 = lax.iota(jnp.int32, 16)
    unit16 = jnp.where(iota16 == 0, 1.0, 0.0).astype(jnp.float32)

    def _zfill(r, _):
        for jb in range(GW // 16):
            h0[r, pl.ds(jb * 16, 16)] = zero16
        return 0
    lax.fori_loop(0, CH, _zfill, 0)

    r0 = sid * RPT

    def _zcopy(k, _):
        pltpu.sync_copy(h0, gsh.at[pl.ds(r0 + k * CH, CH)])
        return 0
    lax.fori_loop(0, RPT // CH, _zcopy, 0)
    pltpu.sync_copy(h0.at[pl.ds(0, RPT % CH)],
                    gsh.at[pl.ds(r0 + (RPT // CH) * CH, RPT % CH)])

    def _initcnt(r, _):
        h0[r, pl.ds(D, 16)] = unit16
        h1[r, pl.ds(D, 16)] = unit16
        return 0
    lax.fori_loop(0, CH, _initcnt, 0)

    plsc.subcore_barrier()

    def _fire_g(k, j):
        pltpu.make_async_copy(tab_hbm.at[cidx.at[j]], gbufs[k], sgs[k]).start()

    def _wait_g(k, j):
        pltpu.make_async_copy(tab_hbm.at[cidx.at[j]], gbufs[k], sgs[k]).wait()

    himask = jnp.int32(-65536)

    def _super(s, _):
        @pl.when(s > 0)
        def _():
            for k in range(2):
                pltpu.make_async_copy(
                    hbufs[k], gsh.at[colb.at[SUB - 2 + k]], sss[k]).wait()

        srow = wid * NCHUNK + s * SUB
        pltpu.sync_copy(cidx_hbm.at[pl.ds(srow, SUB)], cidx)
        pltpu.sync_copy(col_hbm.at[pl.ds(srow, SUB)], colb)
        _fire_g(0, 0)
        _fire_g(1, 1)

        def _pair(t, _):
            for k in range(2):
                j = 2 * t + k
                gb, hb = gbufs[k], hbufs[k]
                _wait_g(k, j)

                @pl.when(t > 0)
                def _():
                    pltpu.make_async_copy(hb, gsh.at[colb.at[j]], sss[k]).wait()

                def _row(r, _):
                    for g in range(D // 32):
                        wa = plsc.bitcast(gb[r, pl.ds(g * 32, 32)], jnp.int32)
                        wb = plsc.bitcast(gb[CH + r, pl.ds(g * 32, 32)], jnp.int32)
                        we = plsc.bitcast(gb[2 * CH + r, pl.ds(g * 32, 32)], jnp.int32)
                        lo = (plsc.bitcast(wa << 16, jnp.float32)
                              + plsc.bitcast(wb << 16, jnp.float32)
                              + plsc.bitcast(we << 16, jnp.float32))
                        hi = (plsc.bitcast(wa & himask, jnp.float32)
                              + plsc.bitcast(wb & himask, jnp.float32)
                              + plsc.bitcast(we & himask, jnp.float32))
                        hb[r, pl.ds(g * 32, 16)] = jnp.maximum(lo, 0.0)
                        hb[r, pl.ds(g * 32 + 16, 16)] = jnp.maximum(hi, 0.0)
                    return 0
                lax.fori_loop(0, CH, _row, 0)

                pltpu.make_async_copy(hb, gsh.at[colb.at[j]], sss[k]).start(add=True)

                @pl.when(j + 2 < SUB)
                def _():
                    _fire_g(k, j + 2)
            return 0
        lax.fori_loop(0, SUB // 2, _pair, 0)
        return 0
    lax.fori_loop(0, NSUPER, _super, 0)

    for k in range(2):
        pltpu.make_async_copy(hbufs[k], gsh.at[colb.at[SUB - 2 + k]], sss[k]).wait()

    plsc.subcore_barrier()

    def _wb(kk, _):
        rr = r0 + kk * CH
        pltpu.sync_copy(gsh.at[pl.ds(rr, CH)], h0)
        pltpu.sync_copy(h0, out_hbm.at[cid].at[pl.ds(rr, CH)])
        return 0
    lax.fori_loop(0, RPT // CH, _wb, 0)
    rr = r0 + (RPT // CH) * CH
    pltpu.sync_copy(gsh.at[pl.ds(rr, RPT % CH)], h0.at[pl.ds(0, RPT % CH)])
    pltpu.sync_copy(h0.at[pl.ds(0, RPT % CH)], out_hbm.at[cid].at[pl.ds(rr, RPT % CH)])


_sc_scatter = functools.partial(
    pl.kernel,
    out_type=jax.ShapeDtypeStruct((NC, N, GW), jnp.float32),
    mesh=plsc.VectorSubcoreMesh(core_axis_name="c", subcore_axis_name="s"),
    compiler_params=pltpu.CompilerParams(use_tc_tiling_on_sc=False,
                                         needs_layout_passes=False),
    scratch_types=[
        pltpu.VMEM((SUB, 3 * CH), jnp.int32),
        pltpu.VMEM((SUB, CH), jnp.int32),
        pltpu.VMEM((3 * CH, D), jnp.bfloat16),
        pltpu.VMEM((3 * CH, D), jnp.bfloat16),
        pltpu.VMEM((CH, GW), jnp.float32),
        pltpu.VMEM((CH, GW), jnp.float32),
        pltpu.VMEM_SHARED((N, GW), jnp.float32),
        pltpu.SemaphoreType.DMA,
        pltpu.SemaphoreType.DMA,
        pltpu.SemaphoreType.DMA,
        pltpu.SemaphoreType.DMA,
    ],
)(_sc_body)


def kernel(x, edge_index, edge_attr, We1, be1, We2, be2, Wn1, bn1, Wn2, bn2):
    row = edge_index[0].astype(jnp.int32).reshape(E // CH, CH)
    col = edge_index[1].astype(jnp.int32).reshape(E // CH, CH)
    eid = (jnp.arange(E, dtype=jnp.int32) + 2 * N).reshape(E // CH, CH)
    cidx = jnp.concatenate([row, col + N, eid], axis=1)

    ps = jnp.asarray(_PS)
    wfull = jnp.concatenate([We1[:D, :][:, ps], We1[D:2 * D, :][:, ps],
                             We1[2 * D:, :][:, ps]], axis=1)
    we2e = jnp.zeros((GW, D), jnp.float32).at[:D].set(We2).at[D].set(be2)
    wn1a = Wn1[:D, :]
    wn1b = Wn1[D:, :]

    nxa = N // NB

    def _xmap(i):
        return (jnp.where(i < nxa, i, jnp.where(i < 2 * nxa, i - nxa, 0)), 0)

    def _eamap(i):
        return (jnp.maximum(i - 2 * nxa, 0), 0)

    tab = pl.pallas_call(
        _pre_body,
        grid=(TROWS // NB,),
        in_specs=[pl.BlockSpec((NB, D), _xmap),
                  pl.BlockSpec((NB, D), _eamap),
                  pl.BlockSpec((D, 3 * D), lambda i: (0, 0)),
                  pl.BlockSpec((1, D), lambda i: (0, 0))],
        out_specs=pl.BlockSpec((NB, D), lambda i: (i, 0)),
        out_shape=jax.ShapeDtypeStruct((TROWS, D), jnp.bfloat16),
    )(x, edge_attr, wfull, be1[ps].reshape(1, D))

    gp = _sc_scatter(cidx, col, tab)

    new_x = pl.pallas_call(
        _node_body,
        grid=(N // NB,),
        in_specs=[pl.BlockSpec((NB, GW), lambda i: (i, 0)),
                  pl.BlockSpec((NB, GW), lambda i: (i, 0)),
                  pl.BlockSpec((NB, D), lambda i: (i, 0)),
                  pl.BlockSpec((GW, D), lambda i: (0, 0)),
                  pl.BlockSpec((D, D), lambda i: (0, 0)),
                  pl.BlockSpec((D, D), lambda i: (0, 0)),
                  pl.BlockSpec((1, D), lambda i: (0, 0)),
                  pl.BlockSpec((D, D), lambda i: (0, 0)),
                  pl.BlockSpec((1, D), lambda i: (0, 0))],
        out_specs=pl.BlockSpec((NB, D), lambda i: (i, 0)),
        out_shape=jax.ShapeDtypeStruct((N, D), jnp.float32),
    )(gp[0], gp[1], x, we2e, wn1a, wn1b, bn1.reshape(1, D),
      Wn2, bn2.reshape(1, D))

    return new_x


# speedup vs baseline: 1.0156x; 1.0156x over previous
import functools

import jax
import jax.numpy as jnp
import numpy as np
from jax import lax
from jax.experimental import pallas as pl
from jax.experimental.pallas import tpu as pltpu
from jax.experimental.pallas import tpu_sc as plsc

N = 10000
E = 320000
D = 128
GW = 144
NC = 2
NS = 16
NW = NC * NS
EPW = E // NW
CH = 40
SUB = 10
NCHUNK = EPW // CH
NSUPER = NCHUNK // SUB
RPT = N // NS

_PS = np.empty((D,), np.int64)
for _g in range(D // 32):
    for _k in range(16):
        _PS[_g * 32 + 2 * _k] = _g * 32 + _k
        _PS[_g * 32 + 2 * _k + 1] = _g * 32 + 16 + _k


def _ea_body(ea_ref, c_ref, b_ref, o_ref):
    o_ref[...] = (jnp.dot(ea_ref[...].astype(jnp.bfloat16),
                          c_ref[...].astype(jnp.bfloat16),
                          preferred_element_type=jnp.float32)
                  + b_ref[...]).astype(jnp.bfloat16)


def _xab_body(x_ref, w_ref, o_ref):
    o_ref[...] = jnp.dot(x_ref[...].astype(jnp.bfloat16),
                         w_ref[...].astype(jnp.bfloat16),
                         preferred_element_type=jnp.float32).astype(jnp.bfloat16)


def _node_body(g0_ref, g1_ref, x_ref, we2e_ref, wn1a_ref, wn1b_ref, bn1_ref,
               wn2_ref, bn2_ref, o_ref):
    g = g0_ref[...] + g1_ref[...]
    agg = jnp.dot(g, we2e_ref[...], preferred_element_type=jnp.float32)
    h2 = jnp.maximum(
        jnp.dot(x_ref[...], wn1a_ref[...], preferred_element_type=jnp.float32)
        + jnp.dot(agg, wn1b_ref[...], preferred_element_type=jnp.float32)
        + bn1_ref[...], 0.0)
    o_ref[...] = jnp.dot(h2, wn2_ref[...],
                         preferred_element_type=jnp.float32) + bn2_ref[...]


def _sc_body(row_hbm, col_hbm, ea_hbm, xa_hbm, xb_hbm, out_hbm,
             idxr, idxc, a0, a1, b0, b1, e0, e1, h0, h1, gsh,
             sem_g0, sem_g1, sem_e0, sem_e1, sem_s0, sem_s1):
    cid = lax.axis_index("c")
    sid = lax.axis_index("s")
    wid = sid * NC + cid

    abufs = (a0, a1)
    bbufs = (b0, b1)
    ebufs = (e0, e1)
    hbufs = (h0, h1)
    sgs = (sem_g0, sem_g1)
    ses = (sem_e0, sem_e1)
    sss = (sem_s0, sem_s1)

    zero16 = jnp.zeros((16,), jnp.float32)
    iota16 = lax.iota(jnp.int32, 16)
    unit16 = jnp.where(iota16 == 0, 1.0, 0.0).astype(jnp.float32)

    def _zfill(r, _):
        for jb in range(GW // 16):
            h0[r, pl.ds(jb * 16, 16)] = zero16
        return 0
    lax.fori_loop(0, CH, _zfill, 0)

    r0 = sid * RPT

    def _zcopy(k, _):
        pltpu.sync_copy(h0, gsh.at[pl.ds(r0 + k * CH, CH)])
        return 0
    lax.fori_loop(0, RPT // CH, _zcopy, 0)
    pltpu.sync_copy(h0.at[pl.ds(0, RPT % CH)],
                    gsh.at[pl.ds(r0 + (RPT // CH) * CH, RPT % CH)])

    def _initcnt(r, _):
        h0[r, pl.ds(D, 16)] = unit16
        h1[r, pl.ds(D, 16)] = unit16
        return 0
    lax.fori_loop(0, CH, _initcnt, 0)

    plsc.subcore_barrier()

    def _gparts(k, j):
        return ((xa_hbm.at[idxr.at[j].at[pl.ds(0, 16)]], abufs[k].at[pl.ds(0, 16)]),
                (xa_hbm.at[idxr.at[j].at[pl.ds(16, 24)]], abufs[k].at[pl.ds(16, 24)]),
                (xb_hbm.at[idxc.at[j].at[pl.ds(0, 16)]], bbufs[k].at[pl.ds(0, 16)]),
                (xb_hbm.at[idxc.at[j].at[pl.ds(16, 24)]], bbufs[k].at[pl.ds(16, 24)]))

    def _fire_g(k, j):
        for src, dst in _gparts(k, j):
            pltpu.make_async_copy(src, dst, sgs[k]).start()

    def _wait_g(k, j):
        for src, dst in _gparts(k, j):
            pltpu.make_async_copy(src, dst, sgs[k]).wait()

    def _fire_e(k, c):
        pltpu.make_async_copy(
            ea_hbm.at[pl.ds((wid * NCHUNK + c) * CH, CH)], ebufs[k], ses[k]).start()

    def _wait_e(k, c):
        pltpu.make_async_copy(
            ea_hbm.at[pl.ds((wid * NCHUNK + c) * CH, CH)], ebufs[k], ses[k]).wait()

    _fire_e(0, 0)
    himask = jnp.int32(-65536)

    def _super(s, _):
        @pl.when(s > 0)
        def _():
            for k in range(2):
                pltpu.make_async_copy(
                    hbufs[k], gsh.at[idxc.at[SUB - 2 + k]], sss[k]).wait()

        srow = wid * NCHUNK + s * SUB
        pltpu.sync_copy(row_hbm.at[pl.ds(srow, SUB)], idxr)
        pltpu.sync_copy(col_hbm.at[pl.ds(srow, SUB)], idxc)
        _fire_g(0, 0)
        _fire_g(1, 1)

        def _pair(t, _):
            for k in range(2):
                j = 2 * t + k
                c = s * SUB + j
                ab, bb, eb, hb = abufs[k], bbufs[k], ebufs[k], hbufs[k]
                _wait_g(k, j)
                _wait_e(k, c)

                @pl.when(c + 1 < NCHUNK)
                def _():
                    _fire_e(1 - k, c + 1)

                @pl.when(t > 0)
                def _():
                    pltpu.make_async_copy(hb, gsh.at[idxc.at[j]], sss[k]).wait()

                def _row(r, _):
                    for g in range(D // 32):
                        wa = plsc.bitcast(ab[r, pl.ds(g * 32, 32)], jnp.int32)
                        wb = plsc.bitcast(bb[r, pl.ds(g * 32, 32)], jnp.int32)
                        we = plsc.bitcast(eb[r, pl.ds(g * 32, 32)], jnp.int32)
                        lo = (plsc.bitcast(wa << 16, jnp.float32)
                              + plsc.bitcast(wb << 16, jnp.float32)
                              + plsc.bitcast(we << 16, jnp.float32))
                        hi = (plsc.bitcast(wa & himask, jnp.float32)
                              + plsc.bitcast(wb & himask, jnp.float32)
                              + plsc.bitcast(we & himask, jnp.float32))
                        hb[r, pl.ds(g * 32, 16)] = jnp.maximum(lo, 0.0)
                        hb[r, pl.ds(g * 32 + 16, 16)] = jnp.maximum(hi, 0.0)
                    return 0
                lax.fori_loop(0, CH, _row, 0)

                pltpu.make_async_copy(hb, gsh.at[idxc.at[j]], sss[k]).start(add=True)

                @pl.when(j + 2 < SUB)
                def _():
                    _fire_g(k, j + 2)
            return 0
        lax.fori_loop(0, SUB // 2, _pair, 0)
        return 0
    lax.fori_loop(0, NSUPER, _super, 0)

    for k in range(2):
        pltpu.make_async_copy(hbufs[k], gsh.at[idxc.at[SUB - 2 + k]], sss[k]).wait()

    plsc.subcore_barrier()

    def _wb(kk, _):
        rr = r0 + kk * CH
        pltpu.sync_copy(gsh.at[pl.ds(rr, CH)], h0)
        pltpu.sync_copy(h0, out_hbm.at[cid].at[pl.ds(rr, CH)])
        return 0
    lax.fori_loop(0, RPT // CH, _wb, 0)
    rr = r0 + (RPT // CH) * CH
    pltpu.sync_copy(gsh.at[pl.ds(rr, RPT % CH)], h0.at[pl.ds(0, RPT % CH)])
    pltpu.sync_copy(h0.at[pl.ds(0, RPT % CH)], out_hbm.at[cid].at[pl.ds(rr, RPT % CH)])


_sc_scatter = functools.partial(
    pl.kernel,
    out_type=jax.ShapeDtypeStruct((NC, N, GW), jnp.float32),
    mesh=plsc.VectorSubcoreMesh(core_axis_name="c", subcore_axis_name="s"),
    compiler_params=pltpu.CompilerParams(use_tc_tiling_on_sc=False, needs_layout_passes=False),
    scratch_types=[
        pltpu.VMEM((SUB, CH), jnp.int32),
        pltpu.VMEM((SUB, CH), jnp.int32),
        pltpu.VMEM((CH, D), jnp.bfloat16),
        pltpu.VMEM((CH, D), jnp.bfloat16),
        pltpu.VMEM((CH, D), jnp.bfloat16),
        pltpu.VMEM((CH, D), jnp.bfloat16),
        pltpu.VMEM((CH, D), jnp.bfloat16),
        pltpu.VMEM((CH, D), jnp.bfloat16),
        pltpu.VMEM((CH, GW), jnp.float32),
        pltpu.VMEM((CH, GW), jnp.float32),
        pltpu.VMEM_SHARED((N, GW), jnp.float32),
        pltpu.SemaphoreType.DMA,
        pltpu.SemaphoreType.DMA,
        pltpu.SemaphoreType.DMA,
        pltpu.SemaphoreType.DMA,
        pltpu.SemaphoreType.DMA,
        pltpu.SemaphoreType.DMA,
    ],
)(_sc_body)


def kernel(x, edge_index, edge_attr, We1, be1, We2, be2, Wn1, bn1, Wn2, bn2):
    row = edge_index[0].astype(jnp.int32).reshape(E // CH, CH)
    col = edge_index[1].astype(jnp.int32).reshape(E // CH, CH)

    ps = jnp.asarray(_PS)
    wab = jnp.concatenate([We1[:D, :][:, ps], We1[D:2 * D, :][:, ps]], axis=1)
    wc = We1[2 * D:, :][:, ps]
    we2e = jnp.zeros((GW, D), jnp.float32).at[:D].set(We2).at[D].set(be2)
    wn1a = Wn1[:D, :]
    wn1b = Wn1[D:, :]

    eb = 2000
    nb = 2000

    xab = pl.pallas_call(
        _xab_body,
        grid=(N // nb,),
        in_specs=[pl.BlockSpec((nb, D), lambda i: (i, 0)),
                  pl.BlockSpec((D, 2 * D), lambda i: (0, 0))],
        out_specs=pl.BlockSpec((nb, 2 * D), lambda i: (i, 0)),
        out_shape=jax.ShapeDtypeStruct((N, 2 * D), jnp.bfloat16),
    )(x, wab)

    ea = pl.pallas_call(
        _ea_body,
        grid=(E // eb,),
        in_specs=[pl.BlockSpec((eb, D), lambda i: (i, 0)),
                  pl.BlockSpec((D, D), lambda i: (0, 0)),
                  pl.BlockSpec((1, D), lambda i: (0, 0))],
        out_specs=pl.BlockSpec((eb, D), lambda i: (i, 0)),
        out_shape=jax.ShapeDtypeStruct((E, D), jnp.bfloat16),
    )(edge_attr, wc, be1[ps].reshape(1, D))

    xa = xab[:, :D]
    xb = xab[:, D:]

    gp = _sc_scatter(row, col, ea, xa, xb)

    new_x = pl.pallas_call(
        _node_body,
        grid=(N // nb,),
        in_specs=[pl.BlockSpec((nb, GW), lambda i: (i, 0)),
                  pl.BlockSpec((nb, GW), lambda i: (i, 0)),
                  pl.BlockSpec((nb, D), lambda i: (i, 0)),
                  pl.BlockSpec((GW, D), lambda i: (0, 0)),
                  pl.BlockSpec((D, D), lambda i: (0, 0)),
                  pl.BlockSpec((D, D), lambda i: (0, 0)),
                  pl.BlockSpec((1, D), lambda i: (0, 0)),
                  pl.BlockSpec((D, D), lambda i: (0, 0)),
                  pl.BlockSpec((1, D), lambda i: (0, 0))],
        out_specs=pl.BlockSpec((nb, D), lambda i: (i, 0)),
        out_shape=jax.ShapeDtypeStruct((N, D), jnp.float32),
    )(gp[0], gp[1], x, we2e, wn1a, wn1b, bn1.reshape(1, D),
      Wn2, bn2.reshape(1, D))

    return new_x


# speedup vs baseline: 1.0176x; 1.0020x over previous
import functools

import jax
import jax.numpy as jnp
import numpy as np
from jax import lax
from jax.experimental import pallas as pl
from jax.experimental.pallas import tpu as pltpu
from jax.experimental.pallas import tpu_sc as plsc

N = 10000
E = 320000
D = 128
GW = 144
NC = 2
NS = 16
NW = NC * NS
EPW = E // NW
CH = 40
SUB = 10
NCHUNK = EPW // CH
NSUPER = NCHUNK // SUB
RPT = N // NS

_PS = np.empty((D,), np.int64)
for _g in range(D // 32):
    for _k in range(16):
        _PS[_g * 32 + 2 * _k] = _g * 32 + _k
        _PS[_g * 32 + 2 * _k + 1] = _g * 32 + 16 + _k


def _ea_body(ea_ref, c_ref, b_ref, o_ref):
    o_ref[...] = (jnp.dot(ea_ref[...].astype(jnp.bfloat16),
                          c_ref[...].astype(jnp.bfloat16),
                          preferred_element_type=jnp.float32)
                  + b_ref[...]).astype(jnp.bfloat16)


def _xab_body(x_ref, w_ref, o_ref):
    o_ref[...] = jnp.dot(x_ref[...].astype(jnp.bfloat16),
                         w_ref[...].astype(jnp.bfloat16),
                         preferred_element_type=jnp.float32).astype(jnp.bfloat16)


def _node_body(g0_ref, g1_ref, x_ref, we2e_ref, wn1a_ref, wn1b_ref, bn1_ref,
               wn2_ref, bn2_ref, o_ref):
    g = g0_ref[...] + g1_ref[...]
    agg = jnp.dot(g, we2e_ref[...], preferred_element_type=jnp.float32)
    h2 = jnp.maximum(
        jnp.dot(x_ref[...], wn1a_ref[...], preferred_element_type=jnp.float32)
        + jnp.dot(agg, wn1b_ref[...], preferred_element_type=jnp.float32)
        + bn1_ref[...], 0.0)
    o_ref[...] = jnp.dot(h2, wn2_ref[...],
                         preferred_element_type=jnp.float32) + bn2_ref[...]


def _sc_body(row_hbm, col_hbm, ea_hbm, xa_hbm, xb_hbm, out_hbm,
             idxr, idxc, a0, a1, b0, b1, e0, e1, h0, h1, gsh,
             sem_g0, sem_g1, sem_e0, sem_e1, sem_s0, sem_s1):
    cid = lax.axis_index("c")
    sid = lax.axis_index("s")
    wid = sid * NC + cid

    abufs = (a0, a1)
    bbufs = (b0, b1)
    ebufs = (e0, e1)
    hbufs = (h0, h1)
    sgs = (sem_g0, sem_g1)
    ses = (sem_e0, sem_e1)
    sss = (sem_s0, sem_s1)

    zero16 = jnp.zeros((16,), jnp.float32)
    iota16 = lax.iota(jnp.int32, 16)
    unit16 = jnp.where(iota16 == 0, 1.0, 0.0).astype(jnp.float32)

    def _zfill(r, _):
        for jb in range(GW // 16):
            h0[r, pl.ds(jb * 16, 16)] = zero16
        return 0
    lax.fori_loop(0, CH, _zfill, 0)

    r0 = sid * RPT

    def _zcopy(k, _):
        pltpu.sync_copy(h0, gsh.at[pl.ds(r0 + k * CH, CH)])
        return 0
    lax.fori_loop(0, RPT // CH, _zcopy, 0)
    pltpu.sync_copy(h0.at[pl.ds(0, RPT % CH)],
                    gsh.at[pl.ds(r0 + (RPT // CH) * CH, RPT % CH)])

    def _initcnt(r, _):
        h0[r, pl.ds(D, 16)] = unit16
        h1[r, pl.ds(D, 16)] = unit16
        return 0
    lax.fori_loop(0, CH, _initcnt, 0)

    plsc.subcore_barrier()

    def _fire_g(k, j):
        pltpu.make_async_copy(xa_hbm.at[idxr.at[j]], abufs[k], sgs[k]).start()
        pltpu.make_async_copy(xb_hbm.at[idxc.at[j]], bbufs[k], sgs[k]).start()

    def _wait_g(k, j):
        pltpu.make_async_copy(xa_hbm.at[idxr.at[j]], abufs[k], sgs[k]).wait()
        pltpu.make_async_copy(xb_hbm.at[idxc.at[j]], bbufs[k], sgs[k]).wait()

    def _fire_e(k, c):
        pltpu.make_async_copy(
            ea_hbm.at[pl.ds((wid * NCHUNK + c) * CH, CH)], ebufs[k], ses[k]).start()

    def _wait_e(k, c):
        pltpu.make_async_copy(
            ea_hbm.at[pl.ds((wid * NCHUNK + c) * CH, CH)], ebufs[k], ses[k]).wait()

    _fire_e(0, 0)
    himask = jnp.int32(-65536)

    def _super(s, _):
        @pl.when(s > 0)
        def _():
            for k in range(2):
                pltpu.make_async_copy(
                    hbufs[k], gsh.at[idxc.at[SUB - 2 + k]], sss[k]).wait()

        srow = wid * NCHUNK + s * SUB
        pltpu.sync_copy(row_hbm.at[pl.ds(srow, SUB)], idxr)
        pltpu.sync_copy(col_hbm.at[pl.ds(srow, SUB)], idxc)
        _fire_g(0, 0)
        _fire_g(1, 1)

        def _pair(t, _):
            for k in range(2):
                j = 2 * t + k
                c = s * SUB + j
                ab, bb, eb, hb = abufs[k], bbufs[k], ebufs[k], hbufs[k]
                _wait_g(k, j)
                _wait_e(k, c)

                @pl.when(c + 1 < NCHUNK)
                def _():
                    _fire_e(1 - k, c + 1)

                @pl.when(t > 0)
                def _():
                    pltpu.make_async_copy(hb, gsh.at[idxc.at[j]], sss[k]).wait()

                def _row(r, _):
                    for g in range(D // 32):
                        wa = plsc.bitcast(ab[r, pl.ds(g * 32, 32)], jnp.int32)
                        wb = plsc.bitcast(bb[r, pl.ds(g * 32, 32)], jnp.int32)
                        we = plsc.bitcast(eb[r, pl.ds(g * 32, 32)], jnp.int32)
                        lo = (plsc.bitcast(wa << 16, jnp.float32)
                              + plsc.bitcast(wb << 16, jnp.float32)
                              + plsc.bitcast(we << 16, jnp.float32))
                        hi = (plsc.bitcast(wa & himask, jnp.float32)
                              + plsc.bitcast(wb & himask, jnp.float32)
                              + plsc.bitcast(we & himask, jnp.float32))
                        hb[r, pl.ds(g * 32, 16)] = jnp.maximum(lo, 0.0)
                        hb[r, pl.ds(g * 32 + 16, 16)] = jnp.maximum(hi, 0.0)
                    return 0
                lax.fori_loop(0, CH, _row, 0)

                pltpu.make_async_copy(hb, gsh.at[idxc.at[j]], sss[k]).start(add=True)

                @pl.when(j + 2 < SUB)
                def _():
                    _fire_g(k, j + 2)
            return 0
        lax.fori_loop(0, SUB // 2, _pair, 0)
        return 0
    lax.fori_loop(0, NSUPER, _super, 0)

    for k in range(2):
        pltpu.make_async_copy(hbufs[k], gsh.at[idxc.at[SUB - 2 + k]], sss[k]).wait()

    plsc.subcore_barrier()

    def _wb(kk, _):
        rr = r0 + kk * CH
        pltpu.sync_copy(gsh.at[pl.ds(rr, CH)], h0)
        pltpu.sync_copy(h0, out_hbm.at[cid].at[pl.ds(rr, CH)])
        return 0
    lax.fori_loop(0, RPT // CH, _wb, 0)
    rr = r0 + (RPT // CH) * CH
    pltpu.sync_copy(gsh.at[pl.ds(rr, RPT % CH)], h0.at[pl.ds(0, RPT % CH)])
    pltpu.sync_copy(h0.at[pl.ds(0, RPT % CH)], out_hbm.at[cid].at[pl.ds(rr, RPT % CH)])


_sc_scatter = functools.partial(
    pl.kernel,
    out_type=jax.ShapeDtypeStruct((NC, N, GW), jnp.float32),
    mesh=plsc.VectorSubcoreMesh(core_axis_name="c", subcore_axis_name="s"),
    compiler_params=pltpu.CompilerParams(use_tc_tiling_on_sc=False, needs_layout_passes=False),
    scratch_types=[
        pltpu.VMEM((SUB, CH), jnp.int32),
        pltpu.VMEM((SUB, CH), jnp.int32),
        pltpu.VMEM((CH, D), jnp.bfloat16),
        pltpu.VMEM((CH, D), jnp.bfloat16),
        pltpu.VMEM((CH, D), jnp.bfloat16),
        pltpu.VMEM((CH, D), jnp.bfloat16),
        pltpu.VMEM((CH, D), jnp.bfloat16),
        pltpu.VMEM((CH, D), jnp.bfloat16),
        pltpu.VMEM((CH, GW), jnp.float32),
        pltpu.VMEM((CH, GW), jnp.float32),
        pltpu.VMEM_SHARED((N, GW), jnp.float32),
        pltpu.SemaphoreType.DMA,
        pltpu.SemaphoreType.DMA,
        pltpu.SemaphoreType.DMA,
        pltpu.SemaphoreType.DMA,
        pltpu.SemaphoreType.DMA,
        pltpu.SemaphoreType.DMA,
    ],
)(_sc_body)


def kernel(x, edge_index, edge_attr, We1, be1, We2, be2, Wn1, bn1, Wn2, bn2):
    row = edge_index[0].astype(jnp.int32).reshape(E // CH, CH)
    col = edge_index[1].astype(jnp.int32).reshape(E // CH, CH)

    ps = jnp.asarray(_PS)
    wab = jnp.concatenate([We1[:D, :][:, ps], We1[D:2 * D, :][:, ps]], axis=1)
    wc = We1[2 * D:, :][:, ps]
    we2e = jnp.zeros((GW, D), jnp.float32).at[:D].set(We2).at[D].set(be2)
    wn1a = Wn1[:D, :]
    wn1b = Wn1[D:, :]

    eb = 2000
    nb = 2000

    xab = pl.pallas_call(
        _xab_body,
        grid=(N // nb,),
        in_specs=[pl.BlockSpec((nb, D), lambda i: (i, 0)),
                  pl.BlockSpec((D, 2 * D), lambda i: (0, 0))],
        out_specs=pl.BlockSpec((nb, 2 * D), lambda i: (i, 0)),
        out_shape=jax.ShapeDtypeStruct((N, 2 * D), jnp.bfloat16),
    )(x, wab)

    ea = pl.pallas_call(
        _ea_body,
        grid=(E // eb,),
        in_specs=[pl.BlockSpec((eb, D), lambda i: (i, 0)),
                  pl.BlockSpec((D, D), lambda i: (0, 0)),
                  pl.BlockSpec((1, D), lambda i: (0, 0))],
        out_specs=pl.BlockSpec((eb, D), lambda i: (i, 0)),
        out_shape=jax.ShapeDtypeStruct((E, D), jnp.bfloat16),
    )(edge_attr, wc, be1[ps].reshape(1, D))

    xa = xab[:, :D]
    xb = xab[:, D:]

    gp = _sc_scatter(row, col, ea, xa, xb)

    new_x = pl.pallas_call(
        _node_body,
        grid=(N // nb,),
        in_specs=[pl.BlockSpec((nb, GW), lambda i: (i, 0)),
                  pl.BlockSpec((nb, GW), lambda i: (i, 0)),
                  pl.BlockSpec((nb, D), lambda i: (i, 0)),
                  pl.BlockSpec((GW, D), lambda i: (0, 0)),
                  pl.BlockSpec((D, D), lambda i: (0, 0)),
                  pl.BlockSpec((D, D), lambda i: (0, 0)),
                  pl.BlockSpec((1, D), lambda i: (0, 0)),
                  pl.BlockSpec((D, D), lambda i: (0, 0)),
                  pl.BlockSpec((1, D), lambda i: (0, 0))],
        out_specs=pl.BlockSpec((nb, D), lambda i: (i, 0)),
        out_shape=jax.ShapeDtypeStruct((N, D), jnp.float32),
    )(gp[0], gp[1], x, we2e, wn1a, wn1b, bn1.reshape(1, D),
      Wn2, bn2.reshape(1, D))

    return new_x


# speedup vs baseline: 1.0287x; 1.0109x over previous
import functools

import jax
import jax.numpy as jnp
import numpy as np
from jax import lax
from jax.experimental import pallas as pl
from jax.experimental.pallas import tpu as pltpu
from jax.experimental.pallas import tpu_sc as plsc

N = 10000
E = 320000
D = 128
GW = 144
NC = 2
NS = 16
NW = NC * NS
EPW = E // NW
CH = 40
SUB = 10
NCHUNK = EPW // CH
NSUPER = NCHUNK // SUB
RPT = N // NS

_PS = np.empty((D,), np.int64)
for _g in range(D // 32):
    for _k in range(16):
        _PS[_g * 32 + 2 * _k] = _g * 32 + _k
        _PS[_g * 32 + 2 * _k + 1] = _g * 32 + 16 + _k


def _ea_body(ea_ref, c_ref, b_ref, o_ref):
    o_ref[...] = (jnp.dot(ea_ref[...], c_ref[...],
                          preferred_element_type=jnp.float32)
                  + b_ref[...]).astype(jnp.bfloat16)


def _xab_body(x_ref, w_ref, o_ref):
    o_ref[...] = jnp.dot(x_ref[...], w_ref[...],
                         preferred_element_type=jnp.float32).astype(jnp.bfloat16)


def _node_body(g0_ref, g1_ref, x_ref, we2e_ref, wn1a_ref, wn1b_ref, bn1_ref,
               wn2_ref, bn2_ref, o_ref):
    g = g0_ref[...] + g1_ref[...]
    agg = jnp.dot(g, we2e_ref[...], preferred_element_type=jnp.float32)
    h2 = jnp.maximum(
        jnp.dot(x_ref[...], wn1a_ref[...], preferred_element_type=jnp.float32)
        + jnp.dot(agg, wn1b_ref[...], preferred_element_type=jnp.float32)
        + bn1_ref[...], 0.0)
    o_ref[...] = jnp.dot(h2, wn2_ref[...],
                         preferred_element_type=jnp.float32) + bn2_ref[...]


def _sc_body(row_hbm, col_hbm, ea_hbm, xa_hbm, xb_hbm, out_hbm,
             idxr, idxc, a0, a1, b0, b1, e0, e1, h0, h1, gsh,
             sem_g0, sem_g1, sem_e0, sem_e1, sem_s0, sem_s1):
    cid = lax.axis_index("c")
    sid = lax.axis_index("s")
    wid = sid * NC + cid

    abufs = (a0, a1)
    bbufs = (b0, b1)
    ebufs = (e0, e1)
    hbufs = (h0, h1)
    sgs = (sem_g0, sem_g1)
    ses = (sem_e0, sem_e1)
    sss = (sem_s0, sem_s1)

    zero16 = jnp.zeros((16,), jnp.float32)
    iota16 = lax.iota(jnp.int32, 16)
    unit16 = jnp.where(iota16 == 0, 1.0, 0.0).astype(jnp.float32)

    def _zfill(r, _):
        for jb in range(GW // 16):
            h0[r, pl.ds(jb * 16, 16)] = zero16
        return 0
    lax.fori_loop(0, CH, _zfill, 0)

    r0 = sid * RPT

    def _zcopy(k, _):
        pltpu.sync_copy(h0, gsh.at[pl.ds(r0 + k * CH, CH)])
        return 0
    lax.fori_loop(0, RPT // CH, _zcopy, 0)
    pltpu.sync_copy(h0.at[pl.ds(0, RPT % CH)],
                    gsh.at[pl.ds(r0 + (RPT // CH) * CH, RPT % CH)])

    def _initcnt(r, _):
        h0[r, pl.ds(D, 16)] = unit16
        h1[r, pl.ds(D, 16)] = unit16
        return 0
    lax.fori_loop(0, CH, _initcnt, 0)

    plsc.subcore_barrier()

    def _fire_g(k, j):
        pltpu.make_async_copy(xa_hbm.at[idxr.at[j]], abufs[k], sgs[k]).start()
        pltpu.make_async_copy(xb_hbm.at[idxc.at[j]], bbufs[k], sgs[k]).start()

    def _wait_g(k, j):
        pltpu.make_async_copy(xa_hbm.at[idxr.at[j]], abufs[k], sgs[k]).wait()
        pltpu.make_async_copy(xb_hbm.at[idxc.at[j]], bbufs[k], sgs[k]).wait()

    def _fire_e(k, c):
        pltpu.make_async_copy(
            ea_hbm.at[pl.ds((wid * NCHUNK + c) * CH, CH)], ebufs[k], ses[k]).start()

    def _wait_e(k, c):
        pltpu.make_async_copy(
            ea_hbm.at[pl.ds((wid * NCHUNK + c) * CH, CH)], ebufs[k], ses[k]).wait()

    _fire_e(0, 0)
    himask = jnp.int32(-65536)

    def _super(s, _):
        @pl.when(s > 0)
        def _():
            for k in range(2):
                pltpu.make_async_copy(
                    hbufs[k], gsh.at[idxc.at[SUB - 2 + k]], sss[k]).wait()

        srow = wid * NCHUNK + s * SUB
        pltpu.sync_copy(row_hbm.at[pl.ds(srow, SUB)], idxr)
        pltpu.sync_copy(col_hbm.at[pl.ds(srow, SUB)], idxc)
        _fire_g(0, 0)
        _fire_g(1, 1)

        def _pair(t, _):
            for k in range(2):
                j = 2 * t + k
                c = s * SUB + j
                ab, bb, eb, hb = abufs[k], bbufs[k], ebufs[k], hbufs[k]
                _wait_g(k, j)
                _wait_e(k, c)

                @pl.when(c + 1 < NCHUNK)
                def _():
                    _fire_e(1 - k, c + 1)

                @pl.when(t > 0)
                def _():
                    pltpu.make_async_copy(hb, gsh.at[idxc.at[j]], sss[k]).wait()

                def _row(r, _):
                    for g in range(D // 32):
                        wa = plsc.bitcast(ab[r, pl.ds(g * 32, 32)], jnp.int32)
                        wb = plsc.bitcast(bb[r, pl.ds(g * 32, 32)], jnp.int32)
                        we = plsc.bitcast(eb[r, pl.ds(g * 32, 32)], jnp.int32)
                        lo = (plsc.bitcast(wa << 16, jnp.float32)
                              + plsc.bitcast(wb << 16, jnp.float32)
                              + plsc.bitcast(we << 16, jnp.float32))
                        hi = (plsc.bitcast(wa & himask, jnp.float32)
                              + plsc.bitcast(wb & himask, jnp.float32)
                              + plsc.bitcast(we & himask, jnp.float32))
                        hb[r, pl.ds(g * 32, 16)] = jnp.maximum(lo, 0.0)
                        hb[r, pl.ds(g * 32 + 16, 16)] = jnp.maximum(hi, 0.0)
                    return 0
                lax.fori_loop(0, CH, _row, 0)

                pltpu.make_async_copy(hb, gsh.at[idxc.at[j]], sss[k]).start(add=True)

                @pl.when(j + 2 < SUB)
                def _():
                    _fire_g(k, j + 2)
            return 0
        lax.fori_loop(0, SUB // 2, _pair, 0)
        return 0
    lax.fori_loop(0, NSUPER, _super, 0)

    for k in range(2):
        pltpu.make_async_copy(hbufs[k], gsh.at[idxc.at[SUB - 2 + k]], sss[k]).wait()

    plsc.subcore_barrier()

    def _wb(kk, _):
        rr = r0 + kk * CH
        pltpu.sync_copy(gsh.at[pl.ds(rr, CH)], h0)
        pltpu.sync_copy(h0, out_hbm.at[cid].at[pl.ds(rr, CH)])
        return 0
    lax.fori_loop(0, RPT // CH, _wb, 0)
    rr = r0 + (RPT // CH) * CH
    pltpu.sync_copy(gsh.at[pl.ds(rr, RPT % CH)], h0.at[pl.ds(0, RPT % CH)])
    pltpu.sync_copy(h0.at[pl.ds(0, RPT % CH)], out_hbm.at[cid].at[pl.ds(rr, RPT % CH)])


_sc_scatter = functools.partial(
    pl.kernel,
    out_type=jax.ShapeDtypeStruct((NC, N, GW), jnp.float32),
    mesh=plsc.VectorSubcoreMesh(core_axis_name="c", subcore_axis_name="s"),
    compiler_params=pltpu.CompilerParams(use_tc_tiling_on_sc=False, needs_layout_passes=False),
    scratch_types=[
        pltpu.VMEM((SUB, CH), jnp.int32),
        pltpu.VMEM((SUB, CH), jnp.int32),
        pltpu.VMEM((CH, D), jnp.bfloat16),
        pltpu.VMEM((CH, D), jnp.bfloat16),
        pltpu.VMEM((CH, D), jnp.bfloat16),
        pltpu.VMEM((CH, D), jnp.bfloat16),
        pltpu.VMEM((CH, D), jnp.bfloat16),
        pltpu.VMEM((CH, D), jnp.bfloat16),
        pltpu.VMEM((CH, GW), jnp.float32),
        pltpu.VMEM((CH, GW), jnp.float32),
        pltpu.VMEM_SHARED((N, GW), jnp.float32),
        pltpu.SemaphoreType.DMA,
        pltpu.SemaphoreType.DMA,
        pltpu.SemaphoreType.DMA,
        pltpu.SemaphoreType.DMA,
        pltpu.SemaphoreType.DMA,
        pltpu.SemaphoreType.DMA,
    ],
)(_sc_body)


def kernel(x, edge_index, edge_attr, We1, be1, We2, be2, Wn1, bn1, Wn2, bn2):
    row = edge_index[0].astype(jnp.int32).reshape(E // CH, CH)
    col = edge_index[1].astype(jnp.int32).reshape(E // CH, CH)

    ps = jnp.asarray(_PS)
    wab = jnp.concatenate([We1[:D, :][:, ps], We1[D:2 * D, :][:, ps]], axis=1)
    wc = We1[2 * D:, :][:, ps]
    we2e = jnp.zeros((GW, D), jnp.float32).at[:D].set(We2).at[D].set(be2)
    wn1a = Wn1[:D, :]
    wn1b = Wn1[D:, :]

    eb = 2000
    nb = 2000

    xab = pl.pallas_call(
        _xab_body,
        grid=(N // nb,),
        in_specs=[pl.BlockSpec((nb, D), lambda i: (i, 0)),
                  pl.BlockSpec((D, 2 * D), lambda i: (0, 0))],
        out_specs=pl.BlockSpec((nb, 2 * D), lambda i: (i, 0)),
        out_shape=jax.ShapeDtypeStruct((N, 2 * D), jnp.bfloat16),
    )(x, wab)

    ea = pl.pallas_call(
        _ea_body,
        grid=(E // eb,),
        in_specs=[pl.BlockSpec((eb, D), lambda i: (i, 0)),
                  pl.BlockSpec((D, D), lambda i: (0, 0)),
                  pl.BlockSpec((1, D), lambda i: (0, 0))],
        out_specs=pl.BlockSpec((eb, D), lambda i: (i, 0)),
        out_shape=jax.ShapeDtypeStruct((E, D), jnp.bfloat16),
    )(edge_attr, wc, be1[ps].reshape(1, D))

    xa = xab[:, :D]
    xb = xab[:, D:]

    gp = _sc_scatter(row, col, ea, xa, xb)

    new_x = pl.pallas_call(
        _node_body,
        grid=(N // nb,),
        in_specs=[pl.BlockSpec((nb, GW), lambda i: (i, 0)),
                  pl.BlockSpec((nb, GW), lambda i: (i, 0)),
                  pl.BlockSpec((nb, D), lambda i: (i, 0)),
                  pl.BlockSpec((GW, D), lambda i: (0, 0)),
                  pl.BlockSpec((D, D), lambda i: (0, 0)),
                  pl.BlockSpec((D, D), lambda i: (0, 0)),
                  pl.BlockSpec((1, D), lambda i: (0, 0)),
                  pl.BlockSpec((D, D), lambda i: (0, 0)),
                  pl.BlockSpec((1, D), lambda i: (0, 0))],
        out_specs=pl.BlockSpec((nb, D), lambda i: (i, 0)),
        out_shape=jax.ShapeDtypeStruct((N, D), jnp.float32),
    )(gp[0], gp[1], x, we2e, wn1a, wn1b, bn1.reshape(1, D),
      Wn2, bn2.reshape(1, D))

    return new_x


# speedup vs baseline: 1.0290x; 1.0002x over previous
import functools

import jax
import jax.numpy as jnp
import numpy as np
from jax import lax
from jax.experimental import pallas as pl
from jax.experimental.pallas import tpu as pltpu
from jax.experimental.pallas import tpu_sc as plsc

N = 10000
E = 320000
D = 128
GW = 144
NC = 2
NS = 16
NW = NC * NS
EPW = E // NW
CH = 40
SUB = 10
NCHUNK = EPW // CH
NSUPER = NCHUNK // SUB
RPT = N // NS

_PS = np.empty((D,), np.int64)
for _g in range(D // 32):
    for _k in range(16):
        _PS[_g * 32 + 2 * _k] = _g * 32 + _k
        _PS[_g * 32 + 2 * _k + 1] = _g * 32 + 16 + _k


def _ea_body(ea_ref, c_ref, b_ref, o_ref):
    o_ref[...] = (jnp.dot(ea_ref[...], c_ref[...],
                          preferred_element_type=jnp.float32)
                  + b_ref[...]).astype(jnp.bfloat16)


def _xab_body(x_ref, w_ref, o_ref):
    o_ref[...] = jnp.dot(x_ref[...], w_ref[...],
                         preferred_element_type=jnp.float32).astype(jnp.bfloat16)


def _node_body(g0_ref, g1_ref, x_ref, we2e_ref, wn1a_ref, wn1b_ref, bn1_ref,
               wn2_ref, bn2_ref, o_ref):
    g = g0_ref[...] + g1_ref[...]
    agg = jnp.dot(g, we2e_ref[...], preferred_element_type=jnp.float32)
    h2 = jnp.maximum(
        jnp.dot(x_ref[...], wn1a_ref[...], preferred_element_type=jnp.float32)
        + jnp.dot(agg, wn1b_ref[...], preferred_element_type=jnp.float32)
        + bn1_ref[...], 0.0)
    o_ref[...] = jnp.dot(h2, wn2_ref[...],
                         preferred_element_type=jnp.float32) + bn2_ref[...]


def _sc_body(row_hbm, col_hbm, ea_hbm, xa_hbm, xb_hbm, out_hbm,
             idxr, idxc, a0, a1, b0, b1, e0, e1, h0, h1, gsh,
             sem_g0, sem_g1, sem_e0, sem_e1, sem_s0, sem_s1):
    cid = lax.axis_index("c")
    sid = lax.axis_index("s")
    wid = sid * NC + cid

    abufs = (a0, a1)
    bbufs = (b0, b1)
    ebufs = (e0, e1)
    hbufs = (h0, h1)
    sgs = (sem_g0, sem_g1)
    ses = (sem_e0, sem_e1)
    sss = (sem_s0, sem_s1)

    zero16 = jnp.zeros((16,), jnp.float32)
    iota16 = lax.iota(jnp.int32, 16)
    unit16 = jnp.where(iota16 == 0, 1.0, 0.0).astype(jnp.float32)

    def _zfill(r, _):
        for jb in range(GW // 16):
            h0[r, pl.ds(jb * 16, 16)] = zero16
        return 0
    lax.fori_loop(0, CH, _zfill, 0)

    r0 = sid * RPT

    def _zcopy(k, _):
        pltpu.sync_copy(h0, gsh.at[pl.ds(r0 + k * CH, CH)])
        return 0
    lax.fori_loop(0, RPT // CH, _zcopy, 0)
    pltpu.sync_copy(h0.at[pl.ds(0, RPT % CH)],
                    gsh.at[pl.ds(r0 + (RPT // CH) * CH, RPT % CH)])

    def _initcnt(r, _):
        h0[r, pl.ds(D, 16)] = unit16
        h1[r, pl.ds(D, 16)] = unit16
        return 0
    lax.fori_loop(0, CH, _initcnt, 0)

    plsc.subcore_barrier()

    def _fire_g(k, j):
        pltpu.make_async_copy(xa_hbm.at[idxr.at[j]], abufs[k], sgs[k]).start()
        pltpu.make_async_copy(xb_hbm.at[idxc.at[j]], bbufs[k], sgs[k]).start()

    def _wait_g(k, j):
        pltpu.make_async_copy(xa_hbm.at[idxr.at[j]], abufs[k], sgs[k]).wait()
        pltpu.make_async_copy(xb_hbm.at[idxc.at[j]], bbufs[k], sgs[k]).wait()

    def _fire_e(k, c):
        pltpu.make_async_copy(
            ea_hbm.at[pl.ds((wid * NCHUNK + c) * CH, CH)], ebufs[k], ses[k]).start()

    def _wait_e(k, c):
        pltpu.make_async_copy(
            ea_hbm.at[pl.ds((wid * NCHUNK + c) * CH, CH)], ebufs[k], ses[k]).wait()

    _fire_e(0, 0)
    himask = jnp.int32(-65536)

    def _super(s, _):
        @pl.when(s > 0)
        def _():
            for k in range(2):
                pltpu.make_async_copy(
                    hbufs[k], gsh.at[idxc.at[SUB - 2 + k]], sss[k]).wait()

        srow = wid * NCHUNK + s * SUB
        pltpu.sync_copy(row_hbm.at[pl.ds(srow, SUB)], idxr)
        pltpu.sync_copy(col_hbm.at[pl.ds(srow, SUB)], idxc)
        _fire_g(0, 0)
        _fire_g(1, 1)

        def _pair(t, _):
            for k in range(2):
                j = 2 * t + k
                c = s * SUB + j
                ab, bb, eb, hb = abufs[k], bbufs[k], ebufs[k], hbufs[k]
                _wait_g(k, j)
                _wait_e(k, c)

                @pl.when(c + 1 < NCHUNK)
                def _():
                    _fire_e(1 - k, c + 1)

                @pl.when(t > 0)
                def _():
                    pltpu.make_async_copy(hb, gsh.at[idxc.at[j]], sss[k]).wait()

                def _row(r, _):
                    for g in range(D // 32):
                        wa = plsc.bitcast(ab[r, pl.ds(g * 32, 32)], jnp.int32)
                        wb = plsc.bitcast(bb[r, pl.ds(g * 32, 32)], jnp.int32)
                        we = plsc.bitcast(eb[r, pl.ds(g * 32, 32)], jnp.int32)
                        lo = (plsc.bitcast(wa << 16, jnp.float32)
                              + plsc.bitcast(wb << 16, jnp.float32)
                              + plsc.bitcast(we << 16, jnp.float32))
                        hi = (plsc.bitcast(wa & himask, jnp.float32)
                              + plsc.bitcast(wb & himask, jnp.float32)
                              + plsc.bitcast(we & himask, jnp.float32))
                        hb[r, pl.ds(g * 32, 16)] = jnp.maximum(lo, 0.0)
                        hb[r, pl.ds(g * 32 + 16, 16)] = jnp.maximum(hi, 0.0)
                    return 0
                lax.fori_loop(0, CH, _row, 0)

                pltpu.make_async_copy(hb, gsh.at[idxc.at[j]], sss[k]).start(add=True)

                @pl.when(j + 2 < SUB)
                def _():
                    _fire_g(k, j + 2)
            return 0
        lax.fori_loop(0, SUB // 2, _pair, 0)
        return 0
    lax.fori_loop(0, NSUPER, _super, 0)

    for k in range(2):
        pltpu.make_async_copy(hbufs[k], gsh.at[idxc.at[SUB - 2 + k]], sss[k]).wait()

    plsc.subcore_barrier()

    def _wb(kk, _):
        rr = r0 + kk * CH
        pltpu.sync_copy(gsh.at[pl.ds(rr, CH)], h0)
        pltpu.sync_copy(h0, out_hbm.at[cid].at[pl.ds(rr, CH)])
        return 0
    lax.fori_loop(0, RPT // CH, _wb, 0)
    rr = r0 + (RPT // CH) * CH
    pltpu.sync_copy(gsh.at[pl.ds(rr, RPT % CH)], h0.at[pl.ds(0, RPT % CH)])
    pltpu.sync_copy(h0.at[pl.ds(0, RPT % CH)], out_hbm.at[cid].at[pl.ds(rr, RPT % CH)])


def _make_sc_scatter():
  return functools.partial(
      pl.kernel,
      out_type=jax.ShapeDtypeStruct((NC, N, GW), jnp.float32),
      mesh=plsc.VectorSubcoreMesh(core_axis_name="c", subcore_axis_name="s",
                                num_cores=NC, num_subcores=NS),
      compiler_params=pltpu.CompilerParams(use_tc_tiling_on_sc=False, needs_layout_passes=False),
      scratch_types=[
        pltpu.VMEM((SUB, CH), jnp.int32),
        pltpu.VMEM((SUB, CH), jnp.int32),
        pltpu.VMEM((CH, D), jnp.bfloat16),
        pltpu.VMEM((CH, D), jnp.bfloat16),
        pltpu.VMEM((CH, D), jnp.bfloat16),
        pltpu.VMEM((CH, D), jnp.bfloat16),
        pltpu.VMEM((CH, D), jnp.bfloat16),
        pltpu.VMEM((CH, D), jnp.bfloat16),
        pltpu.VMEM((CH, GW), jnp.float32),
        pltpu.VMEM((CH, GW), jnp.float32),
        pltpu.VMEM_SHARED((N, GW), jnp.float32),
        pltpu.SemaphoreType.DMA,
        pltpu.SemaphoreType.DMA,
        pltpu.SemaphoreType.DMA,
        pltpu.SemaphoreType.DMA,
        pltpu.SemaphoreType.DMA,
        pltpu.SemaphoreType.DMA,
      ],
  )(_sc_body)


def kernel(x, edge_index, edge_attr, We1, be1, We2, be2, Wn1, bn1, Wn2, bn2):
    row = edge_index[0].astype(jnp.int32).reshape(E // CH, CH)
    col = edge_index[1].astype(jnp.int32).reshape(E // CH, CH)

    ps = jnp.asarray(_PS)
    wab = jnp.concatenate([We1[:D, :][:, ps], We1[D:2 * D, :][:, ps]], axis=1)
    wc = We1[2 * D:, :][:, ps]
    we2e = jnp.zeros((GW, D), jnp.float32).at[:D].set(We2).at[D].set(be2)
    wn1a = Wn1[:D, :]
    wn1b = Wn1[D:, :]

    eb = 2000
    nb = 2000

    xab = pl.pallas_call(
        _xab_body,
        grid=(N // nb,),
        in_specs=[pl.BlockSpec((nb, D), lambda i: (i, 0)),
                  pl.BlockSpec((D, 2 * D), lambda i: (0, 0))],
        out_specs=pl.BlockSpec((nb, 2 * D), lambda i: (i, 0)),
        out_shape=jax.ShapeDtypeStruct((N, 2 * D), jnp.bfloat16),
    )(x, wab)

    ea = pl.pallas_call(
        _ea_body,
        grid=(E // eb,),
        in_specs=[pl.BlockSpec((eb, D), lambda i: (i, 0)),
                  pl.BlockSpec((D, D), lambda i: (0, 0)),
                  pl.BlockSpec((1, D), lambda i: (0, 0))],
        out_specs=pl.BlockSpec((eb, D), lambda i: (i, 0)),
        out_shape=jax.ShapeDtypeStruct((E, D), jnp.bfloat16),
    )(edge_attr, wc, be1[ps].reshape(1, D))

    xa = xab[:, :D]
    xb = xab[:, D:]

    gp = _make_sc_scatter()(row, col, ea, xa, xb)

    new_x = pl.pallas_call(
        _node_body,
        grid=(N // nb,),
        in_specs=[pl.BlockSpec((nb, GW), lambda i: (i, 0)),
                  pl.BlockSpec((nb, GW), lambda i: (i, 0)),
                  pl.BlockSpec((nb, D), lambda i: (i, 0)),
                  pl.BlockSpec((GW, D), lambda i: (0, 0)),
                  pl.BlockSpec((D, D), lambda i: (0, 0)),
                  pl.BlockSpec((D, D), lambda i: (0, 0)),
                  pl.BlockSpec((1, D), lambda i: (0, 0)),
                  pl.BlockSpec((D, D), lambda i: (0, 0)),
                  pl.BlockSpec((1, D), lambda i: (0, 0))],
        out_specs=pl.BlockSpec((nb, D), lambda i: (i, 0)),
        out_shape=jax.ShapeDtypeStruct((N, D), jnp.float32),
    )(gp[0], gp[1], x, we2e, wn1a, wn1b, bn1.reshape(1, D),
      Wn2, bn2.reshape(1, D))

    return new_x
